# baseline (speedup 1.0000x reference)
import functools

import jax
import jax.numpy as jnp
from jax.experimental import pallas as pl
from jax.experimental.pallas import tpu as pltpu
from jax.experimental.pallas import tpu_sc as plsc

_T_END = 1000000
_M = 128


def _sc_row_lookup(u, idx):
    mesh = plsc.ScalarSubcoreMesh(axis_name="c", num_cores=1)

    @functools.partial(
        pl.kernel,
        mesh=mesh,
        out_type=jax.ShapeDtypeStruct((_M,), jnp.float32),
        scratch_types=[
            pltpu.SMEM((1,), jnp.int32),
        ],
    )
    def k(u_hbm, idx_hbm, out_hbm, idx_s):
        pltpu.sync_copy(idx_hbm, idx_s)
        pltpu.sync_copy(u_hbm.at[idx_s[0]], out_hbm)

    return k(u, idx)


def kernel(u, t):
    t32 = jnp.asarray(t, jnp.int32)
    valid = t32 < _T_END
    idx = jnp.where(valid, t32, 0).reshape(1)
    row = _sc_row_lookup(u, idx)
    return jnp.where(valid, row, jnp.zeros((), jnp.float32))


# speedup vs baseline: 1.0121x; 1.0121x over previous
import functools

import jax
import jax.numpy as jnp
from jax.experimental import pallas as pl
from jax.experimental.pallas import tpu as pltpu
from jax.experimental.pallas import tpu_sc as plsc

_T_END = 1000000
_M = 128


def _sc_row_lookup(u, idx):
    mesh = plsc.ScalarSubcoreMesh(axis_name="c", num_cores=1)

    @functools.partial(
        pl.kernel,
        mesh=mesh,
        out_type=jax.ShapeDtypeStruct((_M,), jnp.float32),
        scratch_types=[
            pltpu.SMEM((1,), jnp.int32),
        ],
    )
    def k(u_hbm, idx_hbm, out_hbm, idx_s):
        pltpu.sync_copy(idx_hbm, idx_s)
        pltpu.sync_copy(u_hbm.at[idx_s[0]], out_hbm)

    return k(u, idx)


def kernel(u, t):
    t32 = jnp.asarray(t, jnp.int32)
    valid = t32 < _T_END
    idx = jnp.clip(t32, 0, _T_END - 1).reshape(1)
    row = _sc_row_lookup(u, idx)
    return jnp.where(valid, row, jnp.zeros((), jnp.float32))
